# baseline (speedup 1.0000x reference)
import jax
import jax.numpy as jnp
from jax import lax
from jax.experimental import pallas as pl
from jax.experimental.pallas import tpu as pltpu
from jax.experimental.pallas import tpu_sc as plsc

_BIG = 100000.0
_T = 100
_TP = 128
_DIM = 32
_ROWS = 10000
_BC = 2048
_TAIL = _ROWS - (_ROWS // _BC) * _BC
_NW = 8
_BPW = _TP // _NW
_GPAD = 100096
_COLS = 100000
_INV_SQRT2 = 0.7071067811865476


def _sc_gather_body(gpad_hbm, idx_hbm, gc_out, idxv, igv, grows_v, gcol_v,
                    sem):
    wid = lax.axis_index("s") * 2 + lax.axis_index("c")

    @pl.when(wid < _NW)
    def _():
        base = wid * _BPW
        pltpu.sync_copy(idx_hbm.at[pl.ds(base, _BPW)], idxv)
        iv = idxv[...]
        igv[...] = lax.shift_right_logical(iv, 7)
        pltpu.async_copy(gpad_hbm.at[igv], grows_v, sem).wait()
        rowi = lax.iota(jnp.int32, 16)
        gcol_v[...] = plsc.load_gather(grows_v, [rowi, iv & 127])
        pltpu.sync_copy(gcol_v, gc_out.at[pl.ds(base, _BPW)])


def _sc_gather(gpad2, idx):
    return pl.kernel(
        _sc_gather_body,
        out_type=jax.ShapeDtypeStruct((_TP,), jnp.float32),
        mesh=plsc.VectorSubcoreMesh(core_axis_name="c", subcore_axis_name="s"),
        compiler_params=pltpu.CompilerParams(needs_layout_passes=False),
        scratch_types=[
            pltpu.VMEM((_BPW,), jnp.int32),
            pltpu.VMEM((_BPW,), jnp.int32),
            pltpu.VMEM((_BPW, 128), jnp.float32),
            pltpu.VMEM((_BPW,), jnp.float32),
            pltpu.SemaphoreType.DMA,
        ],
    )(gpad2, idx)


def _tc_body(matT_ref, trow_ref, gc_ref, zrT_ref, gr_ref, zcolsT_hbm,
             idx_ref, b_ref, sigma_ref, out_ref, zcT_vmem, gcoarse_vmem,
             zsem):
    nlast = _COLS - 128
    copies = []
    for j in range(_TP):
        cb = jnp.minimum((idx_ref[j] >> 7) * 128, nlast)
        copies.append(pltpu.make_async_copy(
            zcolsT_hbm.at[:, pl.ds(pl.multiple_of(cb, 128), 128)],
            gcoarse_vmem.at[j], zsem))
    for c in copies:
        c.start()
    for c in copies:
        c.wait()
    lane = lax.broadcasted_iota(jnp.int32, (_DIM, 128), 1)
    for j in range(_TP):
        cb = jnp.minimum((idx_ref[j] >> 7) * 128, nlast)
        m = idx_ref[j] - cb
        col = jnp.sum(jnp.where(lane == m, gcoarse_vmem[j], 0.0),
                      axis=1, keepdims=True)
        zcT_vmem[:, pl.ds(j, 1)] = col

    tr1 = trow_ref[...]
    tr2 = 0.5 * tr1 * tr1
    wvT = zcT_vmem[...] - 1e-6
    ww = jnp.sum(wvT * wvT, axis=0, keepdims=True)
    W = jnp.concatenate([
        -2.0 * wvT, (-2.0 * tr1) * wvT, (-2.0 * tr2) * wvT,
    ], axis=0)
    qcoef = [jnp.ones((_TP, 1), jnp.float32),
             jnp.transpose(2.0 * tr1), jnp.transpose(2.0 * tr2),
             jnp.transpose(tr1 * tr1), jnp.transpose(2.0 * tr1 * tr2),
             jnp.transpose(tr2 * tr2)]
    ww_col = jnp.transpose(ww)

    gc_col = gc_ref[...][:_T]
    th = [-_BIG, b_ref[0], b_ref[1], b_ref[2], b_ref[3], _BIG]
    inv_sigma = 1.0 / sigma_ref[0]

    def chunk_ll(off, w):
        z0 = zrT_ref[0, :, pl.ds(off, w)]
        z1 = zrT_ref[1, :, pl.ds(off, w)]
        z2 = zrT_ref[2, :, pl.ds(off, w)]
        FT = jnp.concatenate([z0, z1, z2], axis=0)
        G = lax.dot_general(W, FT, (((0,), (0,)), ((), ())),
                            preferred_element_type=jnp.float32,
                            precision=None)
        qs = [jnp.sum(z0 * z0, axis=0, keepdims=True),
              jnp.sum(z0 * z1, axis=0, keepdims=True),
              jnp.sum(z0 * z2, axis=0, keepdims=True),
              jnp.sum(z1 * z1, axis=0, keepdims=True),
              jnp.sum(z1 * z2, axis=0, keepdims=True),
              jnp.sum(z2 * z2, axis=0, keepdims=True)]
        dist2 = (G + ww_col
                 + qcoef[0] * qs[0] + qcoef[1] * qs[1] + qcoef[2] * qs[2]
                 + qcoef[3] * qs[3] + qcoef[4] * qs[4] + qcoef[5] * qs[5])
        dist = jnp.sqrt(jnp.maximum(dist2[:_T], 0.0))
        f = -dist + gc_col + gr_ref[:, pl.ds(off, w)]
        mat = matT_ref[:, pl.ds(off, w)]
        active = mat != 0
        y1 = jnp.where(active, mat, 1)
        thi = jnp.where(y1 == 1, th[1],
              jnp.where(y1 == 2, th[2],
              jnp.where(y1 == 3, th[3], th[4])))
        tlo = jnp.where(y1 == 1, th[0],
              jnp.where(y1 == 2, th[1],
              jnp.where(y1 == 3, th[2], th[3])))
        cdf_hi = 0.5 * (1.0 + lax.erf((thi - f) * inv_sigma * _INV_SQRT2))
        cdf_lo = 0.5 * (1.0 + lax.erf((tlo - f) * inv_sigma * _INV_SQRT2))
        ll = jnp.log(cdf_hi - cdf_lo)
        return jnp.sum(jnp.where(active, ll, 0.0))

    def body(k, acc):
        off = pl.multiple_of(k * _BC, 128)
        return acc + chunk_ll(off, _BC)

    acc = lax.fori_loop(0, _ROWS // _BC, body, jnp.float32(0.0))
    acc = acc + chunk_ll(pl.multiple_of((_ROWS // _BC) * _BC, 128), _TAIL)
    out_ref[0, 0] = acc


def _tc_call(matT, trow, gc_col, z_rowsT, gr_row, z_colsT, idx, b, sigma):
    return pl.pallas_call(
        _tc_body,
        grid=(1,),
        in_specs=[
            pl.BlockSpec((_T, _ROWS), lambda i: (0, 0)),
            pl.BlockSpec((1, _TP), lambda i: (0, 0)),
            pl.BlockSpec((_TP, 1), lambda i: (0, 0)),
            pl.BlockSpec((3, _DIM, _ROWS), lambda i: (0, 0, 0)),
            pl.BlockSpec((1, _ROWS), lambda i: (0, 0)),
            pl.BlockSpec(memory_space=pltpu.MemorySpace.HBM),
            pl.BlockSpec(memory_space=pltpu.SMEM),
            pl.BlockSpec(memory_space=pltpu.SMEM),
            pl.BlockSpec(memory_space=pltpu.SMEM),
        ],
        out_specs=pl.BlockSpec((1, 1), lambda i: (0, 0),
                               memory_space=pltpu.SMEM),
        out_shape=jax.ShapeDtypeStruct((1, 1), jnp.float32),
        scratch_shapes=[
            pltpu.VMEM((_DIM, _TP), jnp.float32),
            pltpu.VMEM((_TP, _DIM, 128), jnp.float32),
            pltpu.SemaphoreType.DMA,
        ],
    )(matT, trow, gc_col, z_rowsT, gr_row, z_colsT, idx, b, sigma)


def kernel(batch_events_mat, col_idx_list, batch_events_time,
           gamma_rows, gamma_cols, z_rows, z_cols, b, sigma):
    idx = jnp.pad(col_idx_list.astype(jnp.int32), (0, _TP - _T))
    gpad2 = jnp.pad(gamma_cols, (0, _GPAD - gamma_cols.shape[0])
                    ).reshape(_GPAD // 128, 128)
    gc = _sc_gather(gpad2, idx)
    gc_col = gc.reshape(_TP, 1)
    trow = jnp.pad(batch_events_time, (0, _TP - _T)).reshape(1, _TP)
    gr_row = gamma_rows.reshape(1, -1)
    total = _tc_call(batch_events_mat.T.astype(jnp.int32), trow, gc_col,
                     z_rows.transpose(0, 2, 1), gr_row, z_cols.T, idx,
                     b.astype(jnp.float32), sigma)
    return -total[0, 0]


# speedup vs baseline: 1.1523x; 1.1523x over previous
import jax
import jax.numpy as jnp
from jax import lax
from jax.experimental import pallas as pl
from jax.experimental.pallas import tpu as pltpu
from jax.experimental.pallas import tpu_sc as plsc

_BIG = 100000.0
_T = 100
_TP = 128
_DIM = 32
_ROWS = 10000
_BC = 2048
_TAIL = _ROWS - (_ROWS // _BC) * _BC
_NW = 7
_BPW = 16
_GPAD = 100096
_COLS = 100000
_INV_SQRT2 = 0.7071067811865476


def _sc_gather_body(gpad_hbm, idx_hbm, gc_out, idxv, igv, grows_v, gcol_v,
                    sem):
    wid = lax.axis_index("s") * 2 + lax.axis_index("c")

    @pl.when(wid < _NW)
    def _():
        base = wid * _BPW
        pltpu.sync_copy(idx_hbm.at[pl.ds(base, _BPW)], idxv)
        iv = jnp.clip(idxv[...], 0, _COLS - 1)
        igv[...] = lax.shift_right_logical(iv, 7)
        pltpu.async_copy(gpad_hbm.at[igv], grows_v, sem).wait()
        rowi = lax.iota(jnp.int32, 16)
        gcol_v[...] = plsc.load_gather(grows_v, [rowi, iv & 127])
        pltpu.sync_copy(gcol_v, gc_out.at[pl.ds(base, _BPW)])


def _sc_gather(gpad2, idx):
    return pl.kernel(
        _sc_gather_body,
        out_type=jax.ShapeDtypeStruct((_NW * _BPW,), jnp.float32),
        mesh=plsc.VectorSubcoreMesh(core_axis_name="c", subcore_axis_name="s"),
        compiler_params=pltpu.CompilerParams(needs_layout_passes=False),
        scratch_types=[
            pltpu.VMEM((_BPW,), jnp.int32),
            pltpu.VMEM((_BPW,), jnp.int32),
            pltpu.VMEM((_BPW, 128), jnp.float32),
            pltpu.VMEM((_BPW,), jnp.float32),
            pltpu.SemaphoreType.DMA,
        ],
    )(gpad2, idx)


def _tc_body(matT_ref, trow_ref, gc_ref, zrT_ref, gr_hbm, zcolsT_hbm,
             idx_ref, b_ref, sigma_ref, out_ref, zcT_vmem, gcoarse_vmem,
             gr_vmem, zsem, gsem):
    grdma = pltpu.make_async_copy(gr_hbm, gr_vmem, gsem)
    grdma.start()
    nlast = _COLS - 128
    zcT_vmem[...] = jnp.zeros((_DIM, _TP), jnp.float32)
    copies = []
    for j in range(_T):
        cb = jnp.minimum((idx_ref[j] >> 7) * 128, nlast)
        copies.append(pltpu.make_async_copy(
            zcolsT_hbm.at[:, pl.ds(pl.multiple_of(cb, 128), 128)],
            gcoarse_vmem.at[j], zsem))
    for c in copies:
        c.start()
    for c in copies:
        c.wait()
    lane = lax.broadcasted_iota(jnp.int32, (_DIM, 128), 1)
    for j in range(_T):
        cb = jnp.minimum((idx_ref[j] >> 7) * 128, nlast)
        m = idx_ref[j] - cb
        col = jnp.sum(jnp.where(lane == m, gcoarse_vmem[j], 0.0),
                      axis=1, keepdims=True)
        zcT_vmem[:, pl.ds(j, 1)] = col

    tr1 = trow_ref[...]
    tr2 = 0.5 * tr1 * tr1
    wvT = zcT_vmem[...] - 1e-6
    ww = jnp.sum(wvT * wvT, axis=0, keepdims=True)
    ones_blk = jnp.ones((_DIM, _TP), jnp.float32)
    W = jnp.concatenate([
        -2.0 * wvT, (-2.0 * tr1) * wvT, (-2.0 * tr2) * wvT,
        ones_blk, (2.0 * tr1) * ones_blk, (2.0 * tr2) * ones_blk,
        (tr1 * tr1) * ones_blk, (2.0 * tr1 * tr2) * ones_blk,
        (tr2 * tr2) * ones_blk,
        ww,
    ], axis=0)

    grdma.wait()
    gc_col = gc_ref[...][:_T]
    cs = _INV_SQRT2 / sigma_ref[0]
    th = [-_BIG * cs, b_ref[0] * cs, b_ref[1] * cs, b_ref[2] * cs,
          b_ref[3] * cs, _BIG * cs]

    def chunk_ll(off, w):
        z0 = zrT_ref[0, :, pl.ds(off, w)]
        z1 = zrT_ref[1, :, pl.ds(off, w)]
        z2 = zrT_ref[2, :, pl.ds(off, w)]
        FT = jnp.concatenate([
            z0, z1, z2, z0 * z0, z0 * z1, z0 * z2, z1 * z1, z1 * z2,
            z2 * z2, jnp.ones((1, w), jnp.float32),
        ], axis=0)
        dist2 = lax.dot_general(W, FT, (((0,), (0,)), ((), ())),
                                preferred_element_type=jnp.float32,
                                precision=None)
        dist = jnp.sqrt(jnp.maximum(dist2[:_T], 0.0))
        grw = gr_vmem[pl.ds(off, w)].reshape(1, w)
        fs = (-dist + gc_col + grw) * cs
        mat = matT_ref[:, pl.ds(off, w)]
        active = mat != 0
        y1 = jnp.where(active, mat, 1)
        thi = jnp.where(y1 == 1, th[1],
              jnp.where(y1 == 2, th[2],
              jnp.where(y1 == 3, th[3], th[4])))
        tlo = jnp.where(y1 == 1, th[0],
              jnp.where(y1 == 2, th[1],
              jnp.where(y1 == 3, th[2], th[3])))
        d_cdf = jnp.maximum(lax.erf(thi - fs) - lax.erf(tlo - fs), 0.0)
        ll = jnp.log(0.5 * d_cdf)
        return jnp.sum(jnp.where(active, ll, 0.0))

    def body(k, acc):
        off = pl.multiple_of(k * _BC, 128)
        return acc + chunk_ll(off, _BC)

    acc = lax.fori_loop(0, _ROWS // _BC, body, jnp.float32(0.0))
    acc = acc + chunk_ll(pl.multiple_of((_ROWS // _BC) * _BC, 128), _TAIL)
    out_ref[0, 0] = acc


def _tc_call(matT, trow, gc_col, z_rowsT, gr_row, z_colsT, idx, b, sigma):
    return pl.pallas_call(
        _tc_body,
        grid=(1,),
        in_specs=[
            pl.BlockSpec((_T, _ROWS), lambda i: (0, 0)),
            pl.BlockSpec((1, _TP), lambda i: (0, 0)),
            pl.BlockSpec((_NW * _BPW, 1), lambda i: (0, 0)),
            pl.BlockSpec((3, _DIM, _ROWS), lambda i: (0, 0, 0)),
            pl.BlockSpec(memory_space=pltpu.MemorySpace.HBM),
            pl.BlockSpec(memory_space=pltpu.MemorySpace.HBM),
            pl.BlockSpec(memory_space=pltpu.SMEM),
            pl.BlockSpec(memory_space=pltpu.SMEM),
            pl.BlockSpec(memory_space=pltpu.SMEM),
        ],
        out_specs=pl.BlockSpec((1, 1), lambda i: (0, 0),
                               memory_space=pltpu.SMEM),
        out_shape=jax.ShapeDtypeStruct((1, 1), jnp.float32),
        scratch_shapes=[
            pltpu.VMEM((_DIM, _TP), jnp.float32),
            pltpu.VMEM((_T, _DIM, 128), jnp.float32),
            pltpu.VMEM((_ROWS,), jnp.float32),
            pltpu.SemaphoreType.DMA,
            pltpu.SemaphoreType.DMA,
        ],
    )(matT, trow, gc_col, z_rowsT, gr_row, z_colsT, idx, b, sigma)


def kernel(batch_events_mat, col_idx_list, batch_events_time,
           gamma_rows, gamma_cols, z_rows, z_cols, b, sigma):
    idx = col_idx_list.astype(jnp.int32)
    gpad2 = jnp.pad(gamma_cols, (0, _GPAD - gamma_cols.shape[0])
                    ).reshape(_GPAD // 128, 128)
    gc = _sc_gather(gpad2, idx)
    trow = jnp.pad(batch_events_time, (0, _TP - _T)).reshape(1, _TP)
    total = _tc_call(batch_events_mat.T.astype(jnp.int32), trow,
                     gc.reshape(-1, 1), z_rows.transpose(0, 2, 1),
                     gamma_rows, z_cols.T, idx,
                     b.astype(jnp.float32), sigma)
    return -total[0, 0]
